# Initial kernel scaffold; baseline (speedup 1.0000x reference)
import jax
import jax.numpy as jnp
from jax.experimental import pallas as pl


def kernel(x, edge_index, W, b):
    raise NotImplementedError("write your pallas kernel here")


# speedup vs baseline: 1.6448x; 1.6448x over previous
import functools

import jax
import jax.numpy as jnp
from jax import lax
from jax.experimental import pallas as pl
from jax.experimental.pallas import tpu as pltpu
from jax.experimental.pallas import tpu_sc as plsc

LANES = 16
NW = 32
CN = 8


def _tc_tables(x_t, a_t, b_t, bias):
    np_, c = x_t.shape
    out = a_t.shape[1]

    def body(x_ref, a_ref, bt_ref, bias_ref, u_ref, v_ref):
        xb = x_ref[...]
        u_ref[...] = (
            jnp.dot(xb, a_ref[...], preferred_element_type=jnp.float32)
            + bias_ref[...]
        )
        v_ref[...] = jnp.dot(xb, bt_ref[...], preferred_element_type=jnp.float32)

    return pl.pallas_call(
        body,
        out_shape=[
            jax.ShapeDtypeStruct((np_, out), jnp.float32),
            jax.ShapeDtypeStruct((np_, out), jnp.float32),
        ],
    )(x_t, a_t, b_t, bias)


def _sc_aggregate(u, v, idx_i, idx_j, n_pad, out_dim, k):
    pw = n_pad // NW
    n_chunks = pw // CN
    ce = CN * k
    groups = out_dim // LANES

    mesh = plsc.VectorSubcoreMesh(core_axis_name="c", subcore_axis_name="s")

    @functools.partial(
        pl.kernel,
        mesh=mesh,
        out_type=jax.ShapeDtypeStruct((n_pad, out_dim), jnp.float32),
        scratch_types=[
            pltpu.VMEM((ce,), jnp.int32),
            pltpu.VMEM((ce,), jnp.int32),
            pltpu.VMEM((ce, out_dim), jnp.float32),
            pltpu.VMEM((ce, out_dim), jnp.float32),
            pltpu.VMEM((CN, out_dim), jnp.float32),
            pltpu.SemaphoreType.DMA,
            pltpu.SemaphoreType.DMA,
        ],
    )
    def sc_kernel(u_hbm, v_hbm, ii_hbm, jj_hbm, out_hbm,
                  ii_v, jj_v, u_v, v_v, o_v, sem_u, sem_v):
        wid = lax.axis_index("s") * 2 + lax.axis_index("c")
        base = wid * pw

        def chunk_body(ci, carry):
            ns = base + ci * CN
            es = ns * k
            pltpu.sync_copy(ii_hbm.at[pl.ds(es, ce)], ii_v)
            pltpu.sync_copy(jj_hbm.at[pl.ds(es, ce)], jj_v)
            cp_u = pltpu.async_copy(u_hbm.at[ii_v], u_v, sem_u)
            cp_v = pltpu.async_copy(v_hbm.at[jj_v], v_v, sem_v)
            cp_u.wait()
            cp_v.wait()
            for n in range(CN):
                for g in range(groups):
                    sl = pl.ds(g * LANES, LANES)
                    acc = u_v[n * k, sl] + v_v[n * k, sl]
                    for kk in range(1, k):
                        acc = jnp.maximum(acc, u_v[n * k + kk, sl] + v_v[n * k + kk, sl])
                    o_v[n, sl] = jnp.maximum(acc, 0.0)
            pltpu.sync_copy(o_v, out_hbm.at[pl.ds(ns, CN)])
            return carry

        lax.fori_loop(0, n_chunks, chunk_body, 0)

    return sc_kernel(u, v, idx_i, idx_j)


def kernel(x, edge_index, W, b):
    bb, c, n, _ = x.shape
    k = edge_index.shape[3]
    out_dim = W.shape[0]

    n_pad = ((n + NW * CN - 1) // (NW * CN)) * (NW * CN)

    x_t = jnp.transpose(x.reshape(c, n))
    x_t = jnp.pad(x_t, ((0, n_pad - n), (0, 0)))

    w1 = W[:, :c]
    w2 = W[:, c:]
    a_t = jnp.transpose(w1 - w2)
    b_t = jnp.transpose(w2)
    bias = b.reshape(1, out_dim)

    u, v = _tc_tables(x_t, a_t, b_t, bias)

    ei = edge_index.reshape(2, n * k)
    pad_e = n_pad * k - n * k
    idx_i = jnp.pad(ei[1], (0, pad_e))
    idx_j = jnp.pad(ei[0], (0, pad_e))

    out_full = _sc_aggregate(u, v, idx_i, idx_j, n_pad, out_dim, k)

    out = jnp.transpose(out_full[:n, :]).reshape(bb, out_dim, n, 1)
    return out


# speedup vs baseline: 2.5579x; 1.5552x over previous
import functools

import jax
import jax.numpy as jnp
from jax import lax
from jax.experimental import pallas as pl
from jax.experimental.pallas import tpu as pltpu
from jax.experimental.pallas import tpu_sc as plsc

LANES = 16
NW = 32
CN = 8


def _tc_tables(x_t, a_t, b_t, bias):
    np_, c = x_t.shape
    out = a_t.shape[1]

    def body(x_ref, a_ref, bt_ref, bias_ref, u_ref, v_ref):
        xb = x_ref[...]
        u_ref[...] = (
            jnp.dot(xb, a_ref[...], preferred_element_type=jnp.float32)
            + bias_ref[...]
        )
        v_ref[...] = jnp.dot(xb, bt_ref[...], preferred_element_type=jnp.float32)

    return pl.pallas_call(
        body,
        out_shape=[
            jax.ShapeDtypeStruct((np_, out), jnp.float32),
            jax.ShapeDtypeStruct((np_, out), jnp.float32),
        ],
    )(x_t, a_t, b_t, bias)


def _sc_aggregate(u, v, idx_i, idx_j, n_pad, out_dim, k):
    pw = n_pad // NW
    n_chunks = pw // CN
    ce = CN * k
    groups = out_dim // LANES

    mesh = plsc.VectorSubcoreMesh(core_axis_name="c", subcore_axis_name="s")
    assert n_chunks % 2 == 0

    @functools.partial(
        pl.kernel,
        mesh=mesh,
        out_type=jax.ShapeDtypeStruct((n_pad, out_dim), jnp.float32),
        scratch_types=[
            [pltpu.VMEM((ce,), jnp.int32)] * 2,
            [pltpu.VMEM((ce,), jnp.int32)] * 2,
            [pltpu.VMEM((ce, out_dim), jnp.float32)] * 2,
            [pltpu.VMEM((ce, out_dim), jnp.float32)] * 2,
            [pltpu.VMEM((CN, out_dim), jnp.float32)] * 2,
            [pltpu.SemaphoreType.DMA] * 2,
            [pltpu.SemaphoreType.DMA] * 2,
            [pltpu.SemaphoreType.DMA] * 2,
            [pltpu.SemaphoreType.DMA] * 2,
            [pltpu.SemaphoreType.DMA] * 2,
        ],
    )
    def sc_kernel(u_hbm, v_hbm, ii_hbm, jj_hbm, out_hbm,
                  ii_v, jj_v, u_v, v_v, o_v,
                  sem_ii, sem_jj, sem_u, sem_v, sem_o):
        wid = lax.axis_index("s") * 2 + lax.axis_index("c")
        base = wid * pw

        def idx_start(ci, buf):
            es = (base + ci * CN) * k
            pltpu.make_async_copy(ii_hbm.at[pl.ds(es, ce)], ii_v[buf], sem_ii[buf]).start()
            pltpu.make_async_copy(jj_hbm.at[pl.ds(es, ce)], jj_v[buf], sem_jj[buf]).start()

        def idx_wait(buf):
            pltpu.make_async_copy(ii_hbm.at[pl.ds(0, ce)], ii_v[buf], sem_ii[buf]).wait()
            pltpu.make_async_copy(jj_hbm.at[pl.ds(0, ce)], jj_v[buf], sem_jj[buf]).wait()

        def gather_start(buf):
            pltpu.make_async_copy(u_hbm.at[ii_v[buf]], u_v[buf], sem_u[buf]).start()
            pltpu.make_async_copy(v_hbm.at[jj_v[buf]], v_v[buf], sem_v[buf]).start()

        def gather_wait(buf):
            pltpu.make_async_copy(u_hbm.at[ii_v[buf]], u_v[buf], sem_u[buf]).wait()
            pltpu.make_async_copy(v_hbm.at[jj_v[buf]], v_v[buf], sem_v[buf]).wait()

        idx_start(0, 0)
        idx_start(1, 1)
        idx_wait(0)
        gather_start(0)

        def pair_body(p, carry):
            for b in range(2):
                ci = 2 * p + b
                nxt = ci + 1

                @pl.when(nxt < n_chunks)
                def _():
                    idx_wait(1 - b)
                    gather_start(1 - b)

                gather_wait(b)

                @pl.when(ci + 2 < n_chunks)
                def _():
                    idx_start(ci + 2, b)

                @pl.when(ci >= 2)
                def _():
                    pltpu.make_async_copy(
                        o_v[b], out_hbm.at[pl.ds(base, CN)], sem_o[b]).wait()

                for n in range(CN):
                    for g in range(groups):
                        sl = pl.ds(g * LANES, LANES)
                        acc = u_v[b][n * k, sl] + v_v[b][n * k, sl]
                        for kk in range(1, k):
                            acc = jnp.maximum(
                                acc, u_v[b][n * k + kk, sl] + v_v[b][n * k + kk, sl])
                        o_v[b][n, sl] = jnp.maximum(acc, 0.0)

                ns = base + ci * CN
                pltpu.make_async_copy(o_v[b], out_hbm.at[pl.ds(ns, CN)], sem_o[b]).start()
            return carry

        lax.fori_loop(0, n_chunks // 2, pair_body, 0)

        pltpu.make_async_copy(o_v[0], out_hbm.at[pl.ds(base, CN)], sem_o[0]).wait()
        pltpu.make_async_copy(o_v[1], out_hbm.at[pl.ds(base, CN)], sem_o[1]).wait()

    return sc_kernel(u, v, idx_i, idx_j)


def kernel(x, edge_index, W, b):
    bb, c, n, _ = x.shape
    k = edge_index.shape[3]
    out_dim = W.shape[0]

    n_pad = ((n + NW * CN - 1) // (NW * CN)) * (NW * CN)

    x_t = jnp.transpose(x.reshape(c, n))
    x_t = jnp.pad(x_t, ((0, n_pad - n), (0, 0)))

    w1 = W[:, :c]
    w2 = W[:, c:]
    a_t = jnp.transpose(w1 - w2)
    b_t = jnp.transpose(w2)
    bias = b.reshape(1, out_dim)

    u, v = _tc_tables(x_t, a_t, b_t, bias)

    ei = edge_index.reshape(2, n * k)
    pad_e = n_pad * k - n * k
    idx_i = jnp.pad(ei[1], (0, pad_e))
    idx_j = jnp.pad(ei[0], (0, pad_e))

    out_full = _sc_aggregate(u, v, idx_i, idx_j, n_pad, out_dim, k)

    out = jnp.transpose(out_full[:n, :]).reshape(bb, out_dim, n, 1)
    return out


# speedup vs baseline: 2.7363x; 1.0697x over previous
import functools

import jax
import jax.numpy as jnp
from jax import lax
from jax.experimental import pallas as pl
from jax.experimental.pallas import tpu as pltpu
from jax.experimental.pallas import tpu_sc as plsc

LANES = 16
NW = 32
CN = 4


def _tc_tables(x_t, a_t, b_t, bias):
    np_, c = x_t.shape
    out = a_t.shape[1]

    def body(x_ref, a_ref, bt_ref, bias_ref, u_ref, v_ref):
        xb = x_ref[...]
        u_ref[...] = (
            jnp.dot(xb, a_ref[...], preferred_element_type=jnp.float32)
            + bias_ref[...]
        )
        v_ref[...] = jnp.dot(xb, bt_ref[...], preferred_element_type=jnp.float32)

    return pl.pallas_call(
        body,
        out_shape=[
            jax.ShapeDtypeStruct((np_, out), jnp.float32),
            jax.ShapeDtypeStruct((np_, out), jnp.float32),
        ],
    )(x_t, a_t, b_t, bias)


def _sc_aggregate(u, v, idx_i, idx_j, n_pad, out_dim, k):
    pw = n_pad // NW
    n_chunks = pw // CN
    ce = CN * k
    groups = out_dim // LANES

    mesh = plsc.VectorSubcoreMesh(core_axis_name="c", subcore_axis_name="s")
    D = 3

    @functools.partial(
        pl.kernel,
        mesh=mesh,
        out_type=jax.ShapeDtypeStruct((n_pad, out_dim), jnp.float32),
        scratch_types=[
            [pltpu.VMEM((ce,), jnp.int32)] * D,
            [pltpu.VMEM((ce,), jnp.int32)] * D,
            [pltpu.VMEM((ce, out_dim), jnp.float32)] * D,
            [pltpu.VMEM((ce, out_dim), jnp.float32)] * D,
            [pltpu.VMEM((CN, out_dim), jnp.float32)] * D,
            [pltpu.SemaphoreType.DMA] * D,
            [pltpu.SemaphoreType.DMA] * D,
            [pltpu.SemaphoreType.DMA] * D,
            [pltpu.SemaphoreType.DMA] * D,
            [pltpu.SemaphoreType.DMA] * D,
        ],
    )
    def sc_kernel(u_hbm, v_hbm, ii_hbm, jj_hbm, out_hbm,
                  ii_v, jj_v, u_v, v_v, o_v,
                  sem_ii, sem_jj, sem_u, sem_v, sem_o):
        wid = lax.axis_index("s") * 2 + lax.axis_index("c")
        base = wid * pw

        def idx_start(ci, buf):
            es = (base + ci * CN) * k
            pltpu.make_async_copy(ii_hbm.at[pl.ds(es, ce)], ii_v[buf], sem_ii[buf]).start()
            pltpu.make_async_copy(jj_hbm.at[pl.ds(es, ce)], jj_v[buf], sem_jj[buf]).start()

        def idx_wait(buf):
            pltpu.make_async_copy(ii_hbm.at[pl.ds(0, ce)], ii_v[buf], sem_ii[buf]).wait()
            pltpu.make_async_copy(jj_hbm.at[pl.ds(0, ce)], jj_v[buf], sem_jj[buf]).wait()

        def gather_start(buf):
            pltpu.make_async_copy(u_hbm.at[ii_v[buf]], u_v[buf], sem_u[buf]).start()
            pltpu.make_async_copy(v_hbm.at[jj_v[buf]], v_v[buf], sem_v[buf]).start()

        def gather_wait(buf):
            pltpu.make_async_copy(u_hbm.at[ii_v[buf]], u_v[buf], sem_u[buf]).wait()
            pltpu.make_async_copy(v_hbm.at[jj_v[buf]], v_v[buf], sem_v[buf]).wait()

        for d in range(D):
            idx_start(d, d)
        for d in range(D - 1):
            idx_wait(d)
            gather_start(d)

        def iteration(ci, b):
            @pl.when(ci + D - 1 < n_chunks)
            def _():
                idx_wait((b + D - 1) % D)
                gather_start((b + D - 1) % D)

            gather_wait(b)

            @pl.when(ci + D < n_chunks)
            def _():
                idx_start(ci + D, b)

            @pl.when(ci >= D)
            def _():
                pltpu.make_async_copy(
                    o_v[b], out_hbm.at[pl.ds(base, CN)], sem_o[b]).wait()

            for n in range(CN):
                for g in range(groups):
                    sl = pl.ds(g * LANES, LANES)
                    acc = u_v[b][n * k, sl] + v_v[b][n * k, sl]
                    for kk in range(1, k):
                        acc = jnp.maximum(
                            acc, u_v[b][n * k + kk, sl] + v_v[b][n * k + kk, sl])
                    o_v[b][n, sl] = jnp.maximum(acc, 0.0)

            ns = base + ci * CN
            pltpu.make_async_copy(o_v[b], out_hbm.at[pl.ds(ns, CN)], sem_o[b]).start()

        def body(p, carry):
            for j in range(D):
                iteration(p * D + j, j)
            return carry

        lax.fori_loop(0, n_chunks // D, body, 0)
        for ci in range((n_chunks // D) * D, n_chunks):
            iteration(ci, ci % D)

        for d in range(D):
            pltpu.make_async_copy(o_v[d], out_hbm.at[pl.ds(base, CN)], sem_o[d]).wait()

    return sc_kernel(u, v, idx_i, idx_j)


def kernel(x, edge_index, W, b):
    bb, c, n, _ = x.shape
    k = edge_index.shape[3]
    out_dim = W.shape[0]

    n_pad = ((n + NW * CN - 1) // (NW * CN)) * (NW * CN)

    x_t = jnp.transpose(x.reshape(c, n))
    x_t = jnp.pad(x_t, ((0, n_pad - n), (0, 0)))

    w1 = W[:, :c]
    w2 = W[:, c:]
    a_t = jnp.transpose(w1 - w2)
    b_t = jnp.transpose(w2)
    bias = b.reshape(1, out_dim)

    u, v = _tc_tables(x_t, a_t, b_t, bias)

    ei = edge_index.reshape(2, n * k)
    pad_e = n_pad * k - n * k
    idx_i = jnp.pad(ei[1], (0, pad_e))
    idx_j = jnp.pad(ei[0], (0, pad_e))

    out_full = _sc_aggregate(u, v, idx_i, idx_j, n_pad, out_dim, k)

    out = jnp.transpose(out_full[:n, :]).reshape(bb, out_dim, n, 1)
    return out


# speedup vs baseline: 2.8194x; 1.0304x over previous
import functools

import jax
import jax.numpy as jnp
from jax import lax
from jax.experimental import pallas as pl
from jax.experimental.pallas import tpu as pltpu
from jax.experimental.pallas import tpu_sc as plsc

LANES = 16
NW = 32
CN = 4


def _tc_tables(x_t, a_t, b_t, bias):
    np_, c = x_t.shape
    out = a_t.shape[1]

    def body(x_ref, a_ref, bt_ref, bias_ref, u_ref, v_ref):
        xb = x_ref[...]
        u_ref[...] = (
            jnp.dot(xb, a_ref[...], preferred_element_type=jnp.float32)
            + bias_ref[...]
        )
        v_ref[...] = jnp.dot(xb, bt_ref[...], preferred_element_type=jnp.float32)

    return pl.pallas_call(
        body,
        out_shape=[
            jax.ShapeDtypeStruct((np_, out), jnp.float32),
            jax.ShapeDtypeStruct((np_, out), jnp.float32),
        ],
    )(x_t, a_t, b_t, bias)


def _sc_aggregate(u, v, idx_i, idx_j, n_pad, out_dim, k):
    pw = n_pad // NW
    n_chunks = pw // CN
    ce = CN * k
    groups = out_dim // LANES

    mesh = plsc.VectorSubcoreMesh(core_axis_name="c", subcore_axis_name="s")
    D = 3

    @functools.partial(
        pl.kernel,
        mesh=mesh,
        out_type=jax.ShapeDtypeStruct((n_pad, out_dim), jnp.float32),
        scratch_types=[
            [pltpu.VMEM((ce,), jnp.int32)] * D,
            [pltpu.VMEM((ce,), jnp.int32)] * D,
            [pltpu.VMEM((ce, out_dim), jnp.float32)] * D,
            [pltpu.VMEM((ce, out_dim), jnp.float32)] * D,
            [pltpu.VMEM((CN, out_dim), jnp.float32)] * D,
            [pltpu.SemaphoreType.DMA] * D,
            [pltpu.SemaphoreType.DMA] * D,
            [pltpu.SemaphoreType.DMA] * D,
            [pltpu.SemaphoreType.DMA] * D,
            [pltpu.SemaphoreType.DMA] * D,
        ],
    )
    def sc_kernel(u_hbm, v_hbm, ii_hbm, jj_hbm, out_hbm,
                  ii_v, jj_v, u_v, v_v, o_v,
                  sem_ii, sem_jj, sem_u, sem_v, sem_o):
        wid = lax.axis_index("s") * 2 + lax.axis_index("c")
        base = wid * pw

        def idx_start(ci, buf):
            es = (base + ci * CN) * k
            pltpu.make_async_copy(ii_hbm.at[pl.ds(es, ce)], ii_v[buf], sem_ii[buf]).start()
            pltpu.make_async_copy(jj_hbm.at[pl.ds(es, ce)], jj_v[buf], sem_jj[buf]).start()

        def idx_wait(buf):
            pltpu.make_async_copy(ii_hbm.at[pl.ds(0, ce)], ii_v[buf], sem_ii[buf]).wait()
            pltpu.make_async_copy(jj_hbm.at[pl.ds(0, ce)], jj_v[buf], sem_jj[buf]).wait()

        def gather_start(buf):
            pltpu.make_async_copy(u_hbm.at[ii_v[buf]], u_v[buf], sem_u[buf]).start()
            pltpu.make_async_copy(v_hbm.at[jj_v[buf]], v_v[buf], sem_v[buf]).start()

        def gather_wait(buf):
            pltpu.make_async_copy(u_hbm.at[ii_v[buf]], u_v[buf], sem_u[buf]).wait()
            pltpu.make_async_copy(v_hbm.at[jj_v[buf]], v_v[buf], sem_v[buf]).wait()

        for d in range(D):
            idx_start(d, d)
        for d in range(D - 1):
            idx_wait(d)
            gather_start(d)

        def iteration(ci, b):
            @pl.when(ci + D - 1 < n_chunks)
            def _():
                idx_wait((b + D - 1) % D)
                gather_start((b + D - 1) % D)

            gather_wait(b)

            @pl.when(ci + D < n_chunks)
            def _():
                idx_start(ci + D, b)

            @pl.when(ci >= D)
            def _():
                pltpu.make_async_copy(
                    o_v[b], out_hbm.at[pl.ds(base, CN)], sem_o[b]).wait()

            for n in range(CN):
                for g in range(groups):
                    sl = pl.ds(g * LANES, LANES)
                    s = [u_v[b][n * k + kk, sl] + v_v[b][n * k + kk, sl]
                         for kk in range(k)]
                    while len(s) > 1:
                        s = [jnp.maximum(s[2 * i], s[2 * i + 1])
                             for i in range(len(s) // 2)] + s[len(s) & ~1:]
                    o_v[b][n, sl] = jnp.maximum(s[0], 0.0)

            ns = base + ci * CN
            pltpu.make_async_copy(o_v[b], out_hbm.at[pl.ds(ns, CN)], sem_o[b]).start()

        def body(p, carry):
            for j in range(D):
                iteration(p * D + j, j)
            return carry

        lax.fori_loop(0, n_chunks // D, body, 0)
        for ci in range((n_chunks // D) * D, n_chunks):
            iteration(ci, ci % D)

        for d in range(D):
            pltpu.make_async_copy(o_v[d], out_hbm.at[pl.ds(base, CN)], sem_o[d]).wait()

    return sc_kernel(u, v, idx_i, idx_j)


def kernel(x, edge_index, W, b):
    bb, c, n, _ = x.shape
    k = edge_index.shape[3]
    out_dim = W.shape[0]

    n_pad = ((n + NW * CN - 1) // (NW * CN)) * (NW * CN)

    x_t = jnp.transpose(x.reshape(c, n))
    x_t = jnp.pad(x_t, ((0, n_pad - n), (0, 0)))

    w1 = W[:, :c]
    w2 = W[:, c:]
    a_t = jnp.transpose(w1 - w2)
    b_t = jnp.transpose(w2)
    bias = b.reshape(1, out_dim)

    u, v = _tc_tables(x_t, a_t, b_t, bias)

    ei = edge_index.reshape(2, n * k)
    pad_e = n_pad * k - n * k
    idx_i = jnp.pad(ei[1], (0, pad_e))
    idx_j = jnp.pad(ei[0], (0, pad_e))

    out_full = _sc_aggregate(u, v, idx_i, idx_j, n_pad, out_dim, k)

    out = jnp.transpose(out_full[:n, :]).reshape(bb, out_dim, n, 1)
    return out
